# baseline (speedup 1.0000x reference)
import functools

import jax
import jax.numpy as jnp
from jax import lax
from jax.experimental import pallas as pl
from jax.experimental.pallas import tpu as pltpu
from jax.experimental.pallas import tpu_sc as plsc

_EMBED_DIM = 64
_SEQ_LEN = 200
_K = 10000.0

_NUM_ROWS = 1000000
_BATCH = 4096
_NUM_WORKERS = 32
_ROWS_TOTAL = _BATCH * _SEQ_LEN
_ROWS_PER_W = _ROWS_TOTAL // _NUM_WORKERS
_CHUNK = 80
_CHUNKS_PER_W = _ROWS_PER_W // _CHUNK
_NG = 5
_NO = 3


def _enc_body(o_ref):
    pos = lax.broadcasted_iota(jnp.int32, (_SEQ_LEN, _EMBED_DIM), 0).astype(
        jnp.float32)
    j = lax.broadcasted_iota(jnp.int32, (_SEQ_LEN, _EMBED_DIM), 1)
    i = (j // 2).astype(jnp.float32)
    denom = jnp.exp(i * (2.0 / _EMBED_DIM) * jnp.log(_K))
    ang = pos / denom
    o_ref[...] = jnp.where(j % 2 == 0, jnp.sin(ang), jnp.cos(ang))


def _make_enc():
    return pl.pallas_call(
        _enc_body,
        out_shape=jax.ShapeDtypeStruct((_SEQ_LEN, _EMBED_DIM), jnp.float32),
    )()


_mesh = plsc.VectorSubcoreMesh(core_axis_name="c", subcore_axis_name="s")

_COLS_BLK = 512


def _detile_body(wt_ref, o_ref):
    t = jnp.swapaxes(wt_ref[...], 0, 1)
    o_ref[...] = jnp.concatenate([t, t], axis=1)


def _tc_detile(wt):
    return pl.pallas_call(
        _detile_body,
        grid=(pl.cdiv(_NUM_ROWS, _COLS_BLK),),
        in_specs=[pl.BlockSpec((_EMBED_DIM, _COLS_BLK), lambda c: (0, c))],
        out_specs=pl.BlockSpec((_COLS_BLK, 128), lambda c: (c, 0)),
        out_shape=jax.ShapeDtypeStruct((_NUM_ROWS, 128), jnp.float32),
    )(wt)


@functools.partial(
    pl.kernel,
    mesh=_mesh,
    out_type=jax.ShapeDtypeStruct((_ROWS_TOTAL, _EMBED_DIM), jnp.float32),
    scratch_types=[
        pltpu.VMEM((_ROWS_PER_W,), jnp.int32),
        pltpu.VMEM((_SEQ_LEN * _EMBED_DIM,), jnp.float32),
        pltpu.VMEM((_NG, _CHUNK, 128), jnp.float32),
        pltpu.VMEM((_NO, _CHUNK, _EMBED_DIM), jnp.float32),
        pltpu.SemaphoreType.DMA((_NG,)),
        pltpu.SemaphoreType.DMA((_NO,)),
    ],
)
def _sc_gather_add(w_hbm, idx_hbm, enc_hbm, out_hbm,
                   idx_v, enc_v, rows_v, obuf_v, gsem, osem):
    nc = 2
    wid = lax.axis_index("s") * nc + lax.axis_index("c")
    row_base = wid * _ROWS_PER_W

    pltpu.sync_copy(idx_hbm.at[pl.ds(row_base, _ROWS_PER_W)], idx_v)
    pltpu.sync_copy(enc_hbm, enc_v)

    def fire_gather(c, bg):
        pltpu.async_copy(
            w_hbm.at[idx_v.at[pl.ds(c * _CHUNK, _CHUNK)]],
            rows_v.at[bg],
            gsem.at[bg],
        )

    def wait_gather(bg):
        pltpu.make_async_copy(
            w_hbm.at[idx_v.at[pl.ds(0, _CHUNK)]],
            rows_v.at[bg],
            gsem.at[bg],
        ).wait()

    def fire_out(c, bo):
        pltpu.async_copy(
            obuf_v.at[bo],
            out_hbm.at[pl.ds(row_base + c * _CHUNK, _CHUNK), :],
            osem.at[bo],
        )

    def wait_out(bo):
        pltpu.make_async_copy(
            obuf_v.at[bo],
            out_hbm.at[pl.ds(row_base, _CHUNK), :],
            osem.at[bo],
        ).wait()

    for b in range(_NG):
        fire_gather(b, b)

    def add_chunk(bg, bo, eoff):
        def body(l, _):
            el = lax.rem(eoff + l, _SEQ_LEN) * _EMBED_DIM
            for d in range(_EMBED_DIM // 16):
                e = enc_v[pl.ds(el + d * 16, 16)]
                r = rows_v[bg, l, pl.ds(d * 16, 16)]
                obuf_v[bo, l, pl.ds(d * 16, 16)] = r + e
            return 0
        lax.fori_loop(0, _CHUNK, body, 0, unroll=4)

    def chunk_body(c, _):
        bg = lax.rem(c, _NG)
        bo = lax.rem(c, _NO)
        eoff = lax.rem(c * _CHUNK, _SEQ_LEN)

        @pl.when(c >= _NO)
        def _():
            wait_out(bo)

        wait_gather(bg)
        add_chunk(bg, bo, eoff)
        fire_out(c, bo)

        @pl.when(c + _NG < _CHUNKS_PER_W)
        def _():
            fire_gather(c + _NG, bg)

        return 0

    lax.fori_loop(0, _CHUNKS_PER_W, chunk_body, 0)
    for b in range(_NO):
        wait_out(b)


def kernel(x, W):
    wp = _tc_detile(jnp.swapaxes(W, 0, 1))
    xf = x.reshape(_ROWS_TOTAL)
    enc = _make_enc().reshape(_SEQ_LEN * _EMBED_DIM)
    out = _sc_gather_add(wp, xf, enc)
    return out.reshape(_BATCH, _SEQ_LEN, _EMBED_DIM)


# speedup vs baseline: 1.5843x; 1.5843x over previous
import functools

import jax
import jax.numpy as jnp
from jax import lax
from jax.experimental import pallas as pl
from jax.experimental.pallas import tpu as pltpu
from jax.experimental.pallas import tpu_sc as plsc

_EMBED_DIM = 64
_SEQ_LEN = 200
_K = 10000.0

_NUM_ROWS = 1000000
_BATCH = 4096
_NUM_WORKERS = 32
_ROWS_TOTAL = _BATCH * _SEQ_LEN
_ROWS_PER_W = _ROWS_TOTAL // _NUM_WORKERS
_CHUNK = 80
_CHUNKS_PER_W = _ROWS_PER_W // _CHUNK
_NG = 5
_NO = 3


def _enc_body(o_ref):
    pos = lax.broadcasted_iota(jnp.int32, (_SEQ_LEN, _EMBED_DIM), 0).astype(
        jnp.float32)
    j = lax.broadcasted_iota(jnp.int32, (_SEQ_LEN, _EMBED_DIM), 1)
    i = (j // 2).astype(jnp.float32)
    denom = jnp.exp(i * (2.0 / _EMBED_DIM) * jnp.log(_K))
    ang = pos / denom
    o_ref[...] = jnp.where(j % 2 == 0, jnp.sin(ang), jnp.cos(ang))


def _make_enc():
    return pl.pallas_call(
        _enc_body,
        out_shape=jax.ShapeDtypeStruct((_SEQ_LEN, _EMBED_DIM), jnp.float32),
    )()


_mesh = plsc.VectorSubcoreMesh(core_axis_name="c", subcore_axis_name="s")

_COLS_BLK = 512


def _detile_body(wt_ref, o_ref):
    t = jnp.swapaxes(wt_ref[...], 0, 1)
    o_ref[...] = jnp.concatenate([t, t], axis=1)


def _tc_detile(wt):
    return pl.pallas_call(
        _detile_body,
        grid=(pl.cdiv(_NUM_ROWS, _COLS_BLK),),
        in_specs=[pl.BlockSpec((_EMBED_DIM, _COLS_BLK), lambda c: (0, c))],
        out_specs=pl.BlockSpec((_COLS_BLK, 128), lambda c: (c, 0)),
        out_shape=jax.ShapeDtypeStruct((_NUM_ROWS, 128), jnp.float32),
    )(wt)


@functools.partial(
    pl.kernel,
    mesh=_mesh,
    out_type=jax.ShapeDtypeStruct((_ROWS_TOTAL, _EMBED_DIM), jnp.float32),
    scratch_types=[
        pltpu.VMEM((_ROWS_PER_W,), jnp.int32),
        pltpu.VMEM((_SEQ_LEN * _EMBED_DIM,), jnp.float32),
        pltpu.VMEM((_NG, _CHUNK, 128), jnp.float32),
        pltpu.VMEM((_NO, _CHUNK, _EMBED_DIM), jnp.float32),
        pltpu.SemaphoreType.DMA((_NG,)),
        pltpu.SemaphoreType.DMA((_NO,)),
    ],
)
def _sc_gather_add(w_hbm, idx_hbm, enc_hbm, out_hbm,
                   idx_v, enc_v, rows_v, obuf_v, gsem, osem):
    nc = 2
    wid = lax.axis_index("s") * nc + lax.axis_index("c")
    row_base = wid * _ROWS_PER_W

    pltpu.sync_copy(idx_hbm.at[pl.ds(row_base, _ROWS_PER_W)], idx_v)
    pltpu.sync_copy(enc_hbm, enc_v)

    def fire_gather(c, bg):
        pltpu.async_copy(
            w_hbm.at[idx_v.at[pl.ds(c * _CHUNK, _CHUNK)]],
            rows_v.at[bg],
            gsem.at[bg],
        )

    def wait_gather(bg):
        pltpu.make_async_copy(
            w_hbm.at[idx_v.at[pl.ds(0, _CHUNK)]],
            rows_v.at[bg],
            gsem.at[bg],
        ).wait()

    def fire_out(c, bo):
        pltpu.async_copy(
            obuf_v.at[bo],
            out_hbm.at[pl.ds(row_base + c * _CHUNK, _CHUNK), :],
            osem.at[bo],
        )

    def wait_out(bo):
        pltpu.make_async_copy(
            obuf_v.at[bo],
            out_hbm.at[pl.ds(row_base, _CHUNK), :],
            osem.at[bo],
        ).wait()

    for b in range(_NG):
        fire_gather(b, b)

    def add_chunk(bg, bo, eoff):
        def body(l, el):
            for d in range(_EMBED_DIM // 16):
                e = enc_v[pl.ds(el + d * 16, 16)]
                r = rows_v[bg, l, pl.ds(d * 16, 16)]
                obuf_v[bo, l, pl.ds(d * 16, 16)] = r + e
            nel = el + _EMBED_DIM
            return lax.select(
                nel >= _SEQ_LEN * _EMBED_DIM, jnp.int32(0), nel)
        lax.fori_loop(0, _CHUNK, body, eoff * _EMBED_DIM, unroll=8)

    def chunk_body(c, _):
        bg = lax.rem(c, _NG)
        bo = lax.rem(c, _NO)
        eoff = lax.rem(c * _CHUNK, _SEQ_LEN)

        @pl.when(c >= _NO)
        def _():
            wait_out(bo)

        wait_gather(bg)
        add_chunk(bg, bo, eoff)
        fire_out(c, bo)

        @pl.when(c + _NG < _CHUNKS_PER_W)
        def _():
            fire_gather(c + _NG, bg)

        return 0

    lax.fori_loop(0, _CHUNKS_PER_W, chunk_body, 0)
    for b in range(_NO):
        wait_out(b)


def kernel(x, W):
    wp = jnp.pad(W, ((0, 0), (0, 128 - _EMBED_DIM)))
    xf = x.reshape(_ROWS_TOTAL)
    enc = _make_enc().reshape(_SEQ_LEN * _EMBED_DIM)
    out = _sc_gather_add(wp, xf, enc)
    return out.reshape(_BATCH, _SEQ_LEN, _EMBED_DIM)


# speedup vs baseline: 1.7713x; 1.1180x over previous
import functools

import jax
import jax.numpy as jnp
from jax import lax
from jax.experimental import pallas as pl
from jax.experimental.pallas import tpu as pltpu
from jax.experimental.pallas import tpu_sc as plsc

_EMBED_DIM = 64
_SEQ_LEN = 200
_K = 10000.0

_NUM_ROWS = 1000000
_BATCH = 4096
_NUM_WORKERS = 32
_ROWS_TOTAL = _BATCH * _SEQ_LEN
_ROWS_PER_W = _ROWS_TOTAL // _NUM_WORKERS
_CHUNK = 80
_CHUNKS_PER_W = _ROWS_PER_W // _CHUNK
_NG = 5
_NO = 3


def _enc_body(o_ref):
    pos = lax.broadcasted_iota(jnp.int32, (_SEQ_LEN, _EMBED_DIM), 0).astype(
        jnp.float32)
    j = lax.broadcasted_iota(jnp.int32, (_SEQ_LEN, _EMBED_DIM), 1)
    i = (j // 2).astype(jnp.float32)
    denom = jnp.exp(i * (2.0 / _EMBED_DIM) * jnp.log(_K))
    ang = pos / denom
    o_ref[...] = jnp.where(j % 2 == 0, jnp.sin(ang), jnp.cos(ang))


def _make_enc():
    return pl.pallas_call(
        _enc_body,
        out_shape=jax.ShapeDtypeStruct((_SEQ_LEN, _EMBED_DIM), jnp.float32),
    )()


_mesh = plsc.VectorSubcoreMesh(core_axis_name="c", subcore_axis_name="s")

_COLS_BLK = 4096


def _detile_body(wt_ref, o_ref):
    eye = jnp.eye(_EMBED_DIM, dtype=jnp.float32)
    t = lax.dot_general(wt_ref[...], eye, (((0,), (0,)), ((), ())),
                        preferred_element_type=jnp.float32)
    o_ref[...] = jnp.concatenate([t, t], axis=1)


def _tc_detile(wt):
    return pl.pallas_call(
        _detile_body,
        grid=(pl.cdiv(_NUM_ROWS, _COLS_BLK),),
        in_specs=[pl.BlockSpec((_EMBED_DIM, _COLS_BLK), lambda c: (0, c))],
        out_specs=pl.BlockSpec((_COLS_BLK, 128), lambda c: (c, 0)),
        out_shape=jax.ShapeDtypeStruct((_NUM_ROWS, 128), jnp.float32),
    )(wt)


@functools.partial(
    pl.kernel,
    mesh=_mesh,
    out_type=jax.ShapeDtypeStruct((_ROWS_TOTAL, _EMBED_DIM), jnp.float32),
    scratch_types=[
        pltpu.VMEM((_ROWS_PER_W,), jnp.int32),
        pltpu.VMEM((_SEQ_LEN * _EMBED_DIM,), jnp.float32),
        pltpu.VMEM((_NG, _CHUNK, 128), jnp.float32),
        pltpu.VMEM((_NO, _CHUNK, _EMBED_DIM), jnp.float32),
        pltpu.SemaphoreType.DMA((_NG,)),
        pltpu.SemaphoreType.DMA((_NO,)),
    ],
)
def _sc_gather_add(w_hbm, idx_hbm, enc_hbm, out_hbm,
                   idx_v, enc_v, rows_v, obuf_v, gsem, osem):
    nc = 2
    wid = lax.axis_index("s") * nc + lax.axis_index("c")
    row_base = wid * _ROWS_PER_W

    pltpu.sync_copy(idx_hbm.at[pl.ds(row_base, _ROWS_PER_W)], idx_v)
    pltpu.sync_copy(enc_hbm, enc_v)

    def fire_gather(c, bg):
        pltpu.async_copy(
            w_hbm.at[idx_v.at[pl.ds(c * _CHUNK, _CHUNK)]],
            rows_v.at[bg],
            gsem.at[bg],
        )

    def wait_gather(bg):
        pltpu.make_async_copy(
            w_hbm.at[idx_v.at[pl.ds(0, _CHUNK)]],
            rows_v.at[bg],
            gsem.at[bg],
        ).wait()

    def fire_out(c, bo):
        pltpu.async_copy(
            obuf_v.at[bo],
            out_hbm.at[pl.ds(row_base + c * _CHUNK, _CHUNK), :],
            osem.at[bo],
        )

    def wait_out(bo):
        pltpu.make_async_copy(
            obuf_v.at[bo],
            out_hbm.at[pl.ds(row_base, _CHUNK), :],
            osem.at[bo],
        ).wait()

    for b in range(_NG):
        fire_gather(b, b)

    def add_chunk(bg, bo, eoff):
        def body(l, el):
            for d in range(_EMBED_DIM // 16):
                e = enc_v[pl.ds(el + d * 16, 16)]
                r = rows_v[bg, l, pl.ds(d * 16, 16)]
                obuf_v[bo, l, pl.ds(d * 16, 16)] = r + e
            nel = el + _EMBED_DIM
            return lax.select(
                nel >= _SEQ_LEN * _EMBED_DIM, jnp.int32(0), nel)
        lax.fori_loop(0, _CHUNK, body, eoff * _EMBED_DIM, unroll=8)

    def chunk_body(c, _):
        bg = lax.rem(c, _NG)
        bo = lax.rem(c, _NO)
        eoff = lax.rem(c * _CHUNK, _SEQ_LEN)

        @pl.when(c >= _NO)
        def _():
            wait_out(bo)

        wait_gather(bg)
        add_chunk(bg, bo, eoff)
        fire_out(c, bo)

        @pl.when(c + _NG < _CHUNKS_PER_W)
        def _():
            fire_gather(c + _NG, bg)

        return 0

    lax.fori_loop(0, _CHUNKS_PER_W, chunk_body, 0)
    for b in range(_NO):
        wait_out(b)


def kernel(x, W):
    wp = _tc_detile(jnp.swapaxes(W, 0, 1))
    xf = x.reshape(_ROWS_TOTAL)
    enc = _make_enc().reshape(_SEQ_LEN * _EMBED_DIM)
    out = _sc_gather_add(wp, xf, enc)
    return out.reshape(_BATCH, _SEQ_LEN, _EMBED_DIM)
